# baseline (speedup 1.0000x reference)
import functools

import jax
import jax.numpy as jnp
from jax import lax
from jax.experimental import pallas as pl
from jax.experimental.pallas import tpu as pltpu
from jax.experimental.pallas import tpu_sc as plsc

N_NODES = 10000
NPAD = 10112
D_FEAT = 128
D_HID = 16
N_CLASSES = 7
N_EDGES = 320000
N_TILES = 32
BATCH = 256
NB = 40
EPT = NB * BATCH
ROWS_PER_SUB = NPAD // 16


def _mm_body(x_ref, w_ref, o_ref):
    o_ref[...] = jnp.dot(x_ref[...], w_ref[...],
                         preferred_element_type=jnp.float32)


def _combine_mm_body(p_ref, y1_ref, w_ref, o_ref):
    h = p_ref[0] + p_ref[1] - y1_ref[...]
    o_ref[...] = jnp.dot(h, w_ref[...], preferred_element_type=jnp.float32)


def _logsoftmax_body(q_ref, y2_ref, o_ref):
    s = q_ref[0] + q_ref[1] - y2_ref[...]
    v = s[:N_NODES, :N_CLASSES]
    m = jnp.max(v, axis=0, keepdims=True)
    lse = jnp.log(jnp.sum(jnp.exp(v - m), axis=0, keepdims=True)) + m
    o_ref[...] = v - lse


def _sc_aggregate(table, src_idx, dst_idx):
    mesh = plsc.VectorSubcoreMesh(core_axis_name="c", subcore_axis_name="s")

    @functools.partial(
        pl.kernel,
        mesh=mesh,
        compiler_params=pltpu.CompilerParams(use_tc_tiling_on_sc=False),
        out_type=jax.ShapeDtypeStruct((2, NPAD, D_HID), jnp.float32),
        scratch_types=[
            pltpu.VMEM((NB + 2, BATCH), jnp.int32),
            pltpu.VMEM((NB, BATCH), jnp.int32),
            pltpu.VMEM((BATCH, D_HID), jnp.float32),
            pltpu.VMEM((BATCH, D_HID), jnp.float32),
            pltpu.VMEM_SHARED((NPAD, D_HID), jnp.float32),
            pltpu.VMEM_SHARED((NPAD, D_HID), jnp.float32),
            pltpu.SemaphoreType.DMA,
            pltpu.SemaphoreType.DMA,
        ],
    )
    def agg(table_hbm, src_hbm, dst_hbm, out_hbm,
            src_v, dst_v, buf0, buf1, tab_sh, acc_sh, sem0, sem1):
        c = lax.axis_index("c")
        s = lax.axis_index("s")
        wid = c * 16 + s
        pltpu.sync_copy(src_hbm.at[wid], src_v)
        pltpu.sync_copy(dst_hbm.at[wid], dst_v)
        rsl = pl.ds(s * ROWS_PER_SUB, ROWS_PER_SUB)
        pltpu.sync_copy(table_hbm.at[rsl], tab_sh.at[rsl])
        pltpu.sync_copy(table_hbm.at[rsl], acc_sh.at[rsl])
        plsc.subcore_barrier()

        pltpu.async_copy(tab_sh.at[src_v.at[0]], buf0, sem0)

        def body(j, carry):
            i0 = 2 * j
            pltpu.async_copy(tab_sh.at[src_v.at[i0 + 1]], buf1, sem1)
            pltpu.make_async_copy(
                tab_sh.at[src_v.at[i0]], buf0, sem0).wait()
            pltpu.sync_copy(buf0, acc_sh.at[dst_v.at[i0]], add=True)
            pltpu.async_copy(tab_sh.at[src_v.at[i0 + 2]], buf0, sem0)
            pltpu.make_async_copy(
                tab_sh.at[src_v.at[i0 + 1]], buf1, sem1).wait()
            pltpu.sync_copy(buf1, acc_sh.at[dst_v.at[i0 + 1]], add=True)
            return carry

        lax.fori_loop(0, NB // 2, body, 0)
        pltpu.make_async_copy(tab_sh.at[src_v.at[NB]], buf0, sem0).wait()
        plsc.subcore_barrier()
        pltpu.sync_copy(acc_sh.at[pl.ds(s * ROWS_PER_SUB, ROWS_PER_SUB)],
                        out_hbm.at[c, pl.ds(s * ROWS_PER_SUB, ROWS_PER_SUB)])

    return agg(table, src_idx, dst_idx)


def kernel(x, edge_index, W1, W2):
    src = edge_index[0].astype(jnp.int32)
    dst = edge_index[1].astype(jnp.int32)
    padlen = N_TILES * EPT - N_EDGES
    pad_dst = N_NODES + (jnp.arange(padlen, dtype=jnp.int32)
                         % (NPAD - N_NODES))
    src_p = jnp.concatenate(
        [src, jnp.zeros((padlen,), jnp.int32)]).reshape(N_TILES, NB, BATCH)
    src_p = jnp.concatenate(
        [src_p, jnp.zeros((N_TILES, 2, BATCH), jnp.int32)], axis=1)
    dst_p = jnp.concatenate([dst, pad_dst]).reshape(N_TILES, NB, BATCH)
    xp = jnp.pad(x, ((0, NPAD - N_NODES), (0, 0)))
    W2p = jnp.zeros((D_HID, D_HID), jnp.float32).at[:, :N_CLASSES].set(W2)

    y1 = pl.pallas_call(
        _mm_body,
        out_shape=jax.ShapeDtypeStruct((NPAD, D_HID), jnp.float32),
    )(xp, W1)
    p = _sc_aggregate(y1, src_p, dst_p)
    y2 = pl.pallas_call(
        _combine_mm_body,
        out_shape=jax.ShapeDtypeStruct((NPAD, D_HID), jnp.float32),
    )(p, y1, W2p)
    q = _sc_aggregate(y2, src_p, dst_p)
    out = pl.pallas_call(
        _logsoftmax_body,
        out_shape=jax.ShapeDtypeStruct((N_NODES, N_CLASSES), jnp.float32),
    )(q, y2)
    return out


# speedup vs baseline: 1.0869x; 1.0869x over previous
import functools

import jax
import jax.numpy as jnp
from jax import lax
from jax.experimental import pallas as pl
from jax.experimental.pallas import tpu as pltpu
from jax.experimental.pallas import tpu_sc as plsc

N_NODES = 10000
NPAD = 10112
D_FEAT = 128
D_HID = 16
N_CLASSES = 7
N_EDGES = 320000
N_TILES = 32
BATCH = 128
NB = 80
EPT = NB * BATCH
ROWS_PER_SUB = NPAD // 16


def _mm_body(x_ref, w_ref, o_ref):
    o_ref[...] = jnp.dot(x_ref[...], w_ref[...],
                         preferred_element_type=jnp.float32)


def _final_body(q_ref, p_ref, y1_ref, w_ref, o_ref):
    z = p_ref[0] + p_ref[1] - y1_ref[...]
    agg = q_ref[0] + q_ref[1] - z
    full = jnp.dot(agg, w_ref[...], preferred_element_type=jnp.float32)
    v = full[:N_NODES, :N_CLASSES]
    m = jnp.max(v, axis=0, keepdims=True)
    lse = jnp.log(jnp.sum(jnp.exp(v - m), axis=0, keepdims=True)) + m
    o_ref[...] = v - lse


def _agg_loop(tab_sh, acc_sh, src_v, dst_v, buf0, buf1, sem0, sem1):
    pltpu.async_copy(tab_sh.at[src_v.at[0]], buf0, sem0)

    def body(j, carry):
        i0 = 2 * j
        pltpu.async_copy(tab_sh.at[src_v.at[i0 + 1]], buf1, sem1)
        pltpu.make_async_copy(tab_sh.at[src_v.at[i0]], buf0, sem0).wait()
        pltpu.sync_copy(buf0, acc_sh.at[dst_v.at[i0]], add=True)
        pltpu.async_copy(tab_sh.at[src_v.at[i0 + 2]], buf0, sem0)
        pltpu.make_async_copy(tab_sh.at[src_v.at[i0 + 1]], buf1, sem1).wait()
        pltpu.sync_copy(buf1, acc_sh.at[dst_v.at[i0 + 1]], add=True)
        return carry

    lax.fori_loop(0, NB // 2, body, 0)
    pltpu.make_async_copy(tab_sh.at[src_v.at[NB]], buf0, sem0).wait()


_SC_SCRATCH = [
    pltpu.VMEM((NB + 2, BATCH), jnp.int32),
    pltpu.VMEM((NB, BATCH), jnp.int32),
    pltpu.VMEM((BATCH, D_HID), jnp.float32),
    pltpu.VMEM((BATCH, D_HID), jnp.float32),
    pltpu.VMEM_SHARED((NPAD, D_HID), jnp.float32),
    pltpu.VMEM_SHARED((NPAD, D_HID), jnp.float32),
    pltpu.SemaphoreType.DMA,
    pltpu.SemaphoreType.DMA,
]

_SC_MESH = dict(core_axis_name="c", subcore_axis_name="s")


def _sc_aggregate(table, src_idx, dst_idx):
    @functools.partial(
        pl.kernel,
        mesh=plsc.VectorSubcoreMesh(**_SC_MESH),
        compiler_params=pltpu.CompilerParams(use_tc_tiling_on_sc=False),
        out_type=jax.ShapeDtypeStruct((2, NPAD, D_HID), jnp.float32),
        scratch_types=_SC_SCRATCH,
    )
    def agg(table_hbm, src_hbm, dst_hbm, out_hbm,
            src_v, dst_v, buf0, buf1, tab_sh, acc_sh, sem0, sem1):
        c = lax.axis_index("c")
        s = lax.axis_index("s")
        wid = c * 16 + s
        pltpu.sync_copy(src_hbm.at[wid], src_v)
        pltpu.sync_copy(dst_hbm.at[wid], dst_v)
        rsl = pl.ds(s * ROWS_PER_SUB, ROWS_PER_SUB)
        pltpu.sync_copy(table_hbm.at[rsl], tab_sh.at[rsl])
        pltpu.sync_copy(table_hbm.at[rsl], acc_sh.at[rsl])
        plsc.subcore_barrier()
        _agg_loop(tab_sh, acc_sh, src_v, dst_v, buf0, buf1, sem0, sem1)
        plsc.subcore_barrier()
        pltpu.sync_copy(acc_sh.at[rsl], out_hbm.at[c, rsl])

    return agg(table, src_idx, dst_idx)


def _sc_combine_aggregate(p, y1, src_idx, dst_idx):
    @functools.partial(
        pl.kernel,
        mesh=plsc.VectorSubcoreMesh(**_SC_MESH),
        compiler_params=pltpu.CompilerParams(use_tc_tiling_on_sc=False),
        out_type=jax.ShapeDtypeStruct((2, NPAD, D_HID), jnp.float32),
        scratch_types=_SC_SCRATCH + [
            pltpu.VMEM((ROWS_PER_SUB, D_HID), jnp.float32),
            pltpu.VMEM((ROWS_PER_SUB, D_HID), jnp.float32),
            pltpu.VMEM((ROWS_PER_SUB, D_HID), jnp.float32),
        ],
    )
    def agg(p_hbm, y1_hbm, src_hbm, dst_hbm, out_hbm,
            src_v, dst_v, buf0, buf1, tab_sh, acc_sh, sem0, sem1,
            z_v, p1_v, y1_v):
        c = lax.axis_index("c")
        s = lax.axis_index("s")
        wid = c * 16 + s
        pltpu.sync_copy(src_hbm.at[wid], src_v)
        pltpu.sync_copy(dst_hbm.at[wid], dst_v)
        rsl = pl.ds(s * ROWS_PER_SUB, ROWS_PER_SUB)
        pltpu.sync_copy(p_hbm.at[0, rsl], z_v)
        pltpu.sync_copy(p_hbm.at[1, rsl], p1_v)
        pltpu.sync_copy(y1_hbm.at[rsl], y1_v)

        def zbody(r, carry):
            z_v[r] = z_v[r] + p1_v[r] - y1_v[r]
            return carry

        lax.fori_loop(0, ROWS_PER_SUB, zbody, 0)
        pltpu.sync_copy(z_v, tab_sh.at[rsl])
        pltpu.sync_copy(z_v, acc_sh.at[rsl])
        plsc.subcore_barrier()
        _agg_loop(tab_sh, acc_sh, src_v, dst_v, buf0, buf1, sem0, sem1)
        plsc.subcore_barrier()
        pltpu.sync_copy(acc_sh.at[rsl], out_hbm.at[c, rsl])

    return agg(p, y1, src_idx, dst_idx)


def kernel(x, edge_index, W1, W2):
    src = edge_index[0].astype(jnp.int32)
    dst = edge_index[1].astype(jnp.int32)
    padlen = N_TILES * EPT - N_EDGES
    pad_dst = N_NODES + (jnp.arange(padlen, dtype=jnp.int32)
                         % (NPAD - N_NODES))
    src_p = jnp.concatenate(
        [src, jnp.zeros((padlen,), jnp.int32)]).reshape(N_TILES, NB, BATCH)
    src_p = jnp.concatenate(
        [src_p, jnp.zeros((N_TILES, 2, BATCH), jnp.int32)], axis=1)
    dst_p = jnp.concatenate([dst, pad_dst]).reshape(N_TILES, NB, BATCH)
    xp = jnp.pad(x, ((0, NPAD - N_NODES), (0, 0)))
    W2p = jnp.zeros((D_HID, D_HID), jnp.float32).at[:, :N_CLASSES].set(W2)

    y1 = pl.pallas_call(
        _mm_body,
        out_shape=jax.ShapeDtypeStruct((NPAD, D_HID), jnp.float32),
    )(xp, W1)
    p = _sc_aggregate(y1, src_p, dst_p)
    q = _sc_combine_aggregate(p, y1, src_p, dst_p)
    out = pl.pallas_call(
        _final_body,
        out_shape=jax.ShapeDtypeStruct((N_NODES, N_CLASSES), jnp.float32),
    )(q, p, y1, W2p)
    return out
